# baseline (speedup 1.0000x reference)
import jax
import jax.numpy as jnp
from jax import lax
from jax.experimental import pallas as pl
from jax.experimental.pallas import tpu as pltpu
from jax.experimental.pallas import tpu_sc as plsc

N = 10000
E = 320000
D = 128
H = 64
NC = 40

NBLK = 1000
EBLK = 1000
SC_NS = 16
SC_B = 80
SC_NW = 10
NWR = N // SC_NW
ZROWS = 200


def _mm(x, w, b):
    return jnp.dot(x, w, preferred_element_type=jnp.float32) + b


def _pre_body(h_ref, wh, bh, aw, ab, bw, bb, dw, db, ew, eb,
              hx_ref, ah_ref, dbt_ref, eht_ref):
    c = pl.program_id(1)
    hx = _mm(h_ref[...], wh[...], bh[...])
    hx_ref[...] = hx
    ah_ref[...] = _mm(hx, aw[...], ab[...])
    dh = _mm(hx, dw[...], db[...])
    bhx = _mm(hx, bw[...], bb[...])
    eh = _mm(hx, ew[...], eb[...])

    eht_ref[...] = eh

    @pl.when(c == 0)
    def _():
        dbt_ref[0, :, 0:H] = dh[:, 0:H]
        dbt_ref[0, :, H:D] = bhx[:, 0:H]

    @pl.when(c == 1)
    def _():
        dbt_ref[0, :, 0:H] = dh[:, H:D]
        dbt_ref[0, :, H:D] = bhx[:, H:D]


def _node_body(hxp_ref, ahp_ref, acc0_ref, acc1_ref, aw, ab, bw, bb, dw, db,
               ew, eb, hx_ref, ah_ref, dbt_ref, eht_ref):
    c = pl.program_id(1)
    num = jnp.concatenate([acc0_ref[:, 0:H], acc1_ref[:, 0:H]], axis=1)
    den = jnp.concatenate([acc0_ref[:, H:D], acc1_ref[:, H:D]], axis=1) + 1e-6
    hx = hxp_ref[...] + jnp.maximum(ahp_ref[...] + num / den, 0.0)
    hx_ref[...] = hx
    ah_ref[...] = _mm(hx, aw[...], ab[...])
    dh = _mm(hx, dw[...], db[...])
    bhx = _mm(hx, bw[...], bb[...])
    eh = _mm(hx, ew[...], eb[...])

    eht_ref[...] = eh

    @pl.when(c == 0)
    def _():
        dbt_ref[0, :, 0:H] = dh[:, 0:H]
        dbt_ref[0, :, H:D] = bhx[:, 0:H]

    @pl.when(c == 1)
    def _():
        dbt_ref[0, :, 0:H] = dh[:, H:D]
        dbt_ref[0, :, H:D] = bhx[:, H:D]


def _ce_body(ex0_ref, ex1_ref, cw, cb, ce_ref):
    x = jnp.concatenate([ex0_ref[...], ex1_ref[...]], axis=1)
    y = _mm(x, cw[...], cb[...])
    ce_ref[0, :, :] = y[:, 0:H]
    ce_ref[1, :, :] = y[:, H:D]


def _post_body(hxp_ref, ahp_ref, acc0_ref, acc1_ref, wo, bo, out_ref):
    num = jnp.concatenate([acc0_ref[:, 0:H], acc1_ref[:, 0:H]], axis=1)
    den = jnp.concatenate([acc0_ref[:, H:D], acc1_ref[:, H:D]], axis=1) + 1e-6
    hx = hxp_ref[...] + jnp.maximum(ahp_ref[...] + num / den, 0.0)
    logits = _mm(hx, wo[...], bo[...])
    mask = lax.broadcasted_iota(jnp.int32, logits.shape, 1) < NC
    neg = jnp.where(mask, logits, -1e30)
    m = jnp.max(neg, axis=1, keepdims=True)
    ez = jnp.where(mask, jnp.exp(logits - m), 0.0)
    lse = jnp.log(jnp.sum(ez, axis=1, keepdims=True))
    out_ref[...] = logits - m - lse


def _wspec():
    return pl.BlockSpec((D, D), lambda i, c: (0, 0))


def _bspec():
    return pl.BlockSpec((1, D), lambda i, c: (0, 0))


@jax.jit
def _k_pre(h, wh, bh, aw, ab, bw, bb, dw, db, ew, eb):
    grid = (N // NBLK, 2)
    return pl.pallas_call(
        _pre_body,
        grid=grid,
        in_specs=[pl.BlockSpec((NBLK, D), lambda i, c: (i, 0))]
        + [_wspec() if k % 2 == 0 else _bspec() for k in range(10)],
        out_specs=[
            pl.BlockSpec((NBLK, D), lambda i, c: (i, 0)),
            pl.BlockSpec((NBLK, D), lambda i, c: (i, 0)),
            pl.BlockSpec((1, NBLK, D), lambda i, c: (c, i, 0)),
            pl.BlockSpec((NBLK, D), lambda i, c: (i, 0)),
        ],
        out_shape=[
            jax.ShapeDtypeStruct((N, D), jnp.float32),
            jax.ShapeDtypeStruct((N, D), jnp.float32),
            jax.ShapeDtypeStruct((2, N, D), jnp.float32),
            jax.ShapeDtypeStruct((N, D), jnp.float32),
        ],
    )(h, wh, bh, aw, ab, bw, bb, dw, db, ew, eb)


@jax.jit
def _k_node(hxp, ahp, acc0, acc1, aw, ab, bw, bb, dw, db, ew, eb):
    grid = (N // NBLK, 2)
    return pl.pallas_call(
        _node_body,
        grid=grid,
        in_specs=[
            pl.BlockSpec((NBLK, D), lambda i, c: (i, 0)),
            pl.BlockSpec((NBLK, D), lambda i, c: (i, 0)),
            pl.BlockSpec((NBLK, D), lambda i, c: (i, 0)),
            pl.BlockSpec((NBLK, D), lambda i, c: (i, 0)),
        ]
        + [_wspec() if k % 2 == 0 else _bspec() for k in range(8)],
        out_specs=[
            pl.BlockSpec((NBLK, D), lambda i, c: (i, 0)),
            pl.BlockSpec((NBLK, D), lambda i, c: (i, 0)),
            pl.BlockSpec((1, NBLK, D), lambda i, c: (c, i, 0)),
            pl.BlockSpec((NBLK, D), lambda i, c: (i, 0)),
        ],
        out_shape=[
            jax.ShapeDtypeStruct((N, D), jnp.float32),
            jax.ShapeDtypeStruct((N, D), jnp.float32),
            jax.ShapeDtypeStruct((2, N, D), jnp.float32),
            jax.ShapeDtypeStruct((N, D), jnp.float32),
        ],
    )(hxp, ahp, acc0, acc1, aw, ab, bw, bb, dw, db, ew, eb)


@jax.jit
def _k_ce(ex0, ex1, cw, cb):
    grid = (E // EBLK,)
    return pl.pallas_call(
        _ce_body,
        grid=grid,
        in_specs=[
            pl.BlockSpec((EBLK, H), lambda i: (i, 0)),
            pl.BlockSpec((EBLK, H), lambda i: (i, 0)),
            pl.BlockSpec((D, D), lambda i: (0, 0)),
            pl.BlockSpec((1, D), lambda i: (0, 0)),
        ],
        out_specs=pl.BlockSpec((2, EBLK, H), lambda i: (0, i, 0)),
        out_shape=jax.ShapeDtypeStruct((2, E, H), jnp.float32),
    )(ex0, ex1, cw, cb)


@jax.jit
def _k_post(hxp, ahp, acc0, acc1, wo, bo):
    grid = (N // NBLK,)
    return pl.pallas_call(
        _post_body,
        grid=grid,
        in_specs=[
            pl.BlockSpec((NBLK, D), lambda i: (i, 0)),
            pl.BlockSpec((NBLK, D), lambda i: (i, 0)),
            pl.BlockSpec((NBLK, D), lambda i: (i, 0)),
            pl.BlockSpec((NBLK, D), lambda i: (i, 0)),
            pl.BlockSpec((D, D), lambda i: (0, 0)),
            pl.BlockSpec((1, D), lambda i: (0, 0)),
        ],
        out_specs=pl.BlockSpec((NBLK, D), lambda i: (i, 0)),
        out_shape=jax.ShapeDtypeStruct((N, D), jnp.float32),
    )(hxp, ahp, acc0, acc1, wo, bo)


def _ex0_body(w_ref, c1_ref, c0_ref, we_ref, be_ref, ce_ref, e0_ref, e1_ref):
    w = w_ref[...]
    ce_ref[0, :, :] = w * c1_ref[0:1, :] + c0_ref[0:1, :]
    ce_ref[1, :, :] = w * c1_ref[1:2, :] + c0_ref[1:2, :]
    e0_ref[...] = w * we_ref[0:1, :] + be_ref[0:1, :]
    e1_ref[...] = w * we_ref[1:2, :] + be_ref[1:2, :]


@jax.jit
def _k_ex0(w, c1, c0, wev, bev):
    grid = (E // EBLK,)
    two64 = pl.BlockSpec((2, H), lambda i: (0, 0))
    return pl.pallas_call(
        _ex0_body,
        grid=grid,
        in_specs=[pl.BlockSpec((EBLK, 1), lambda i: (i, 0)),
                  two64, two64, two64, two64],
        out_specs=[
            pl.BlockSpec((2, EBLK, H), lambda i: (0, i, 0)),
            pl.BlockSpec((EBLK, H), lambda i: (i, 0)),
            pl.BlockSpec((EBLK, H), lambda i: (i, 0)),
        ],
        out_shape=[
            jax.ShapeDtypeStruct((2, E, H), jnp.float32),
            jax.ShapeDtypeStruct((E, H), jnp.float32),
            jax.ShapeDtypeStruct((E, H), jnp.float32),
        ],
    )(w, c1, c0, wev, bev)


NACC = 5008
NEP = 2
SC_NW2 = 5


def _make_sc_layer():
    e_per_s = E // SC_NS
    nblk = e_per_s // SC_B
    nhalf = N // NEP
    nwr = nhalf // SC_NW2
    mesh = plsc.VectorSubcoreMesh(core_axis_name="c", subcore_axis_name="s",
                                  num_cores=1)

    out_type = [jax.ShapeDtypeStruct((N, D), jnp.float32),
                jax.ShapeDtypeStruct((N, D), jnp.float32),
                jax.ShapeDtypeStruct((E, H), jnp.float32),
                jax.ShapeDtypeStruct((E, H), jnp.float32),
                jax.ShapeDtypeStruct((E, D), jnp.float32)]

    scratch = [
        pltpu.VMEM((SC_B,), jnp.int32),
        pltpu.VMEM((SC_B,), jnp.int32),
        pltpu.VMEM((SC_B,), jnp.int32),
        pltpu.VMEM((SC_B, D), jnp.float32),
        pltpu.VMEM((SC_B, D), jnp.float32),
        pltpu.VMEM((SC_B, H), jnp.float32),
        pltpu.VMEM((SC_B, H), jnp.float32),
        pltpu.VMEM((SC_B, D), jnp.float32),
        pltpu.VMEM((ZROWS, D), jnp.float32),
        pltpu.VMEM_SHARED((NACC, D), jnp.float32),
        pltpu.SemaphoreType.DMA,
        pltpu.SemaphoreType.DMA,
    ]

    def body(src_h, dst_h, dbt_h, eht_h, ce_h, ex0_h, ex1_h,
             acc0_h, acc1_h, exo0_h, exo1_h, cs_h,
             src_v, dst_v, idx_v, db_buf, eh_buf, ce_buf, ex_buf,
             cs_buf, zbuf, acc_sp, sem1, sem2):
        s = lax.axis_index("s")
        base_e = s * e_per_s

        def zrow(r, _):
            for k in range(D // 16):
                zbuf[r, pl.ds(16 * k, 16)] = jnp.zeros((16,), jnp.float32)
            return 0

        lax.fori_loop(0, ZROWS, zrow, 0)

        def clamp_idx(lo):
            def idxc(k, _):
                sl = pl.ds(16 * k, 16)
                t = dst_v[sl] - lo
                ok = (t >= 0) & (t < nhalf)
                idx_v[sl] = jnp.where(ok, t, nhalf)
                return 0

            lax.fori_loop(0, SC_B // 16, idxc, 0)

        for half in (0, 1):
            ex_h = ex0_h if half == 0 else ex1_h
            exo_h = exo0_h if half == 0 else exo1_h
            acc_h = acc0_h if half == 0 else acc1_h

            for ep in range(NEP):
                lo = ep * nhalf

                @pl.when(s < SC_NW2)
                def _():
                    for i in range(nwr // ZROWS):
                        pltpu.sync_copy(
                            zbuf, acc_sp.at[pl.ds(s * nwr + ZROWS * i, ZROWS)])

                @pl.when(s == SC_NW2)
                def _():
                    pltpu.sync_copy(zbuf.at[pl.ds(0, 8)],
                                    acc_sp.at[pl.ds(nhalf, 8)])

                plsc.subcore_barrier()

                if ep == 0:
                    def eblk(b, _):
                        e0 = base_e + b * SC_B
                        pltpu.sync_copy(src_h.at[pl.ds(e0, SC_B)], src_v)
                        pltpu.sync_copy(dst_h.at[pl.ds(e0, SC_B)], dst_v)

                        if half == 0:
                            sidx = src_v
                        else:
                            def offc(k, _):
                                sl = pl.ds(16 * k, 16)
                                src_v[sl] = src_v[sl] + N
                                return 0

                            lax.fori_loop(0, SC_B // 16, offc, 0)
                            sidx = src_v
                        cp1 = pltpu.async_copy(dbt_h.at[sidx], db_buf, sem1)
                        cp2 = pltpu.async_copy(eht_h.at[dst_v], eh_buf, sem2)
                        pltpu.sync_copy(ce_h.at[half, pl.ds(e0, SC_B)], ce_buf)
                        pltpu.sync_copy(ex_h.at[pl.ds(e0, SC_B)], ex_buf)
                        clamp_idx(lo)
                        cp1.wait()
                        cp2.wait()

                        def row(r, _):
                            for k in range(H // 16):
                                sl = pl.ds(16 * k, 16)
                                dh = db_buf[r, sl]
                                bh = db_buf[r, pl.ds(H + 16 * k, 16)]
                                en = dh \
                                    + eh_buf[r, pl.ds(half * H + 16 * k, 16)] \
                                    + ce_buf[r, sl]
                                sg = 1.0 / (1.0 + jnp.exp(-en))
                                cs_buf[r, sl] = sg * bh
                                cs_buf[r, pl.ds(H + 16 * k, 16)] = sg
                                ex_buf[r, sl] = ex_buf[r, sl] \
                                    + jnp.maximum(en, 0.0)
                            return 0

                        lax.fori_loop(0, SC_B, row, 0)

                        pltpu.sync_copy(cs_buf, acc_sp.at[idx_v], add=True)
                        pltpu.sync_copy(cs_buf, cs_h.at[pl.ds(e0, SC_B)])
                        pltpu.sync_copy(ex_buf, exo_h.at[pl.ds(e0, SC_B)])
                        return 0
                else:
                    def eblk(b, _):
                        e0 = base_e + b * SC_B
                        pltpu.sync_copy(dst_h.at[pl.ds(e0, SC_B)], dst_v)
                        pltpu.sync_copy(cs_h.at[pl.ds(e0, SC_B)], cs_buf)
                        clamp_idx(lo)
                        pltpu.sync_copy(cs_buf, acc_sp.at[idx_v], add=True)
                        return 0

                lax.fori_loop(0, nblk, eblk, 0)
                plsc.subcore_barrier()

                @pl.when(s < SC_NW2)
                def _():
                    pltpu.sync_copy(acc_sp.at[pl.ds(s * nwr, nwr)],
                                    acc_h.at[pl.ds(lo + s * nwr, nwr)])

                plsc.subcore_barrier()

    return pl.kernel(body, out_type=out_type, mesh=mesh,
                     scratch_types=scratch)


_SC_EDGE = _make_sc_layer()


@jax.jit
def _sc_layer(src, dst, dbt, eht, ce, ex0, ex1):
    a0, a1, eo0, eo1, _ = _SC_EDGE(src, dst, dbt, eht, ce, ex0, ex1)
    return a0, a1, eo0, eo1


def kernel(h, edge_index, edge_weight, W_h, b_h, W_e, b_e, Aw, Ab, Bw, Bb,
           Cw, Cb, Dw, Db, Ew, Eb, W_out, b_out):
    src = edge_index[0]
    dst = edge_index[1]
    r = lambda v: v.reshape(1, D)

    hx0, ah0, dbt0, eht0 = _k_pre(h, W_h, r(b_h), Aw[0], r(Ab[0]), Bw[0],
                                  r(Bb[0]), Dw[0], r(Db[0]), Ew[0], r(Eb[0]))

    wrow = W_e[0]
    c1 = (wrow @ Cw[0]).reshape(2, H)
    c0 = (b_e @ Cw[0] + Cb[0]).reshape(2, H)
    wev = wrow.reshape(2, H)
    bev = b_e.reshape(2, H)
    ce0, ex00, ex01 = _k_ex0(edge_weight.reshape(E, 1), c1, c0, wev, bev)

    acc00, acc01, ex10, ex11 = _sc_layer(src, dst, dbt0.reshape(2 * N, D),
                                         eht0, ce0, ex00, ex01)

    hx1, ah1, dbt1, eht1 = _k_node(hx0, ah0, acc00, acc01, Aw[1], r(Ab[1]),
                                   Bw[1], r(Bb[1]), Dw[1], r(Db[1]),
                                   Ew[1], r(Eb[1]))
    ce1 = _k_ce(ex10, ex11, Cw[1], r(Cb[1]))
    acc10, acc11, ex20, ex21 = _sc_layer(src, dst, dbt1.reshape(2 * N, D),
                                         eht1, ce1, ex10, ex11)

    hx2, ah2, dbt2, eht2 = _k_node(hx1, ah1, acc10, acc11, Aw[2], r(Ab[2]),
                                   Bw[2], r(Bb[2]), Dw[2], r(Db[2]),
                                   Ew[2], r(Eb[2]))
    ce2 = _k_ce(ex20, ex21, Cw[2], r(Cb[2]))
    acc20, acc21, _, _ = _sc_layer(src, dst, dbt2.reshape(2 * N, D),
                                   eht2, ce2, ex20, ex21)

    wo = jnp.pad(W_out, ((0, 0), (0, D - NC)))
    bo = jnp.pad(b_out, (0, D - NC)).reshape(1, D)
    out = _k_post(hx2, ah2, acc20, acc21, wo, bo)
    return out[:, :NC]


# speedup vs baseline: 1.1465x; 1.1465x over previous
import jax
import jax.numpy as jnp
from jax import lax
from jax.experimental import pallas as pl
from jax.experimental.pallas import tpu as pltpu
from jax.experimental.pallas import tpu_sc as plsc

N = 10000
E = 320000
D = 128
H = 64
NC = 40

NBLK = 1000
EBLK = 1000
SC_NS = 16
SC_B = 80
SC_NW = 10
NWR = N // SC_NW
ZROWS = 200


def _mm(x, w, b):
    return jnp.dot(x, w, preferred_element_type=jnp.float32) + b


def _pre_body(h_ref, wh, bh, aw, ab, bw, bb, dw, db, ew, eb,
              hx_ref, ah_ref, dbt_ref, eht_ref):
    c = pl.program_id(1)
    hx = _mm(h_ref[...], wh[...], bh[...])
    hx_ref[...] = hx
    ah_ref[...] = _mm(hx, aw[...], ab[...])
    dh = _mm(hx, dw[...], db[...])
    bhx = _mm(hx, bw[...], bb[...])
    eh = _mm(hx, ew[...], eb[...])

    eht_ref[...] = eh

    @pl.when(c == 0)
    def _():
        dbt_ref[0, :, 0:H] = dh[:, 0:H]
        dbt_ref[0, :, H:D] = bhx[:, 0:H]

    @pl.when(c == 1)
    def _():
        dbt_ref[0, :, 0:H] = dh[:, H:D]
        dbt_ref[0, :, H:D] = bhx[:, H:D]


def _node_body(hxp_ref, ahp_ref, acc0_ref, acc1_ref, aw, ab, bw, bb, dw, db,
               ew, eb, hx_ref, ah_ref, dbt_ref, eht_ref):
    c = pl.program_id(1)
    num = jnp.concatenate([acc0_ref[:, 0:H], acc1_ref[:, 0:H]], axis=1)
    den = jnp.concatenate([acc0_ref[:, H:D], acc1_ref[:, H:D]], axis=1) + 1e-6
    hx = hxp_ref[...] + jnp.maximum(ahp_ref[...] + num / den, 0.0)
    hx_ref[...] = hx
    ah_ref[...] = _mm(hx, aw[...], ab[...])
    dh = _mm(hx, dw[...], db[...])
    bhx = _mm(hx, bw[...], bb[...])
    eh = _mm(hx, ew[...], eb[...])

    eht_ref[...] = eh

    @pl.when(c == 0)
    def _():
        dbt_ref[0, :, 0:H] = dh[:, 0:H]
        dbt_ref[0, :, H:D] = bhx[:, 0:H]

    @pl.when(c == 1)
    def _():
        dbt_ref[0, :, 0:H] = dh[:, H:D]
        dbt_ref[0, :, H:D] = bhx[:, H:D]


def _ce_body(ex0_ref, ex1_ref, cw, cb, ce_ref):
    x = jnp.concatenate([ex0_ref[...], ex1_ref[...]], axis=1)
    y = _mm(x, cw[...], cb[...])
    ce_ref[0, :, :] = y[:, 0:H]
    ce_ref[1, :, :] = y[:, H:D]


def _post_body(hxp_ref, ahp_ref, acc0_ref, acc1_ref, wo, bo, out_ref):
    num = jnp.concatenate([acc0_ref[:, 0:H], acc1_ref[:, 0:H]], axis=1)
    den = jnp.concatenate([acc0_ref[:, H:D], acc1_ref[:, H:D]], axis=1) + 1e-6
    hx = hxp_ref[...] + jnp.maximum(ahp_ref[...] + num / den, 0.0)
    logits = _mm(hx, wo[...], bo[...])
    mask = lax.broadcasted_iota(jnp.int32, logits.shape, 1) < NC
    neg = jnp.where(mask, logits, -1e30)
    m = jnp.max(neg, axis=1, keepdims=True)
    ez = jnp.where(mask, jnp.exp(logits - m), 0.0)
    lse = jnp.log(jnp.sum(ez, axis=1, keepdims=True))
    out_ref[...] = logits - m - lse


def _wspec():
    return pl.BlockSpec((D, D), lambda i, c: (0, 0))


def _bspec():
    return pl.BlockSpec((1, D), lambda i, c: (0, 0))


@jax.jit
def _k_pre(h, wh, bh, aw, ab, bw, bb, dw, db, ew, eb):
    grid = (N // NBLK, 2)
    return pl.pallas_call(
        _pre_body,
        grid=grid,
        in_specs=[pl.BlockSpec((NBLK, D), lambda i, c: (i, 0))]
        + [_wspec() if k % 2 == 0 else _bspec() for k in range(10)],
        out_specs=[
            pl.BlockSpec((NBLK, D), lambda i, c: (i, 0)),
            pl.BlockSpec((NBLK, D), lambda i, c: (i, 0)),
            pl.BlockSpec((1, NBLK, D), lambda i, c: (c, i, 0)),
            pl.BlockSpec((NBLK, D), lambda i, c: (i, 0)),
        ],
        out_shape=[
            jax.ShapeDtypeStruct((N, D), jnp.float32),
            jax.ShapeDtypeStruct((N, D), jnp.float32),
            jax.ShapeDtypeStruct((2, N, D), jnp.float32),
            jax.ShapeDtypeStruct((N, D), jnp.float32),
        ],
    )(h, wh, bh, aw, ab, bw, bb, dw, db, ew, eb)


@jax.jit
def _k_node(hxp, ahp, acc0, acc1, aw, ab, bw, bb, dw, db, ew, eb):
    grid = (N // NBLK, 2)
    return pl.pallas_call(
        _node_body,
        grid=grid,
        in_specs=[
            pl.BlockSpec((NBLK, D), lambda i, c: (i, 0)),
            pl.BlockSpec((NBLK, D), lambda i, c: (i, 0)),
            pl.BlockSpec((NBLK, D), lambda i, c: (i, 0)),
            pl.BlockSpec((NBLK, D), lambda i, c: (i, 0)),
        ]
        + [_wspec() if k % 2 == 0 else _bspec() for k in range(8)],
        out_specs=[
            pl.BlockSpec((NBLK, D), lambda i, c: (i, 0)),
            pl.BlockSpec((NBLK, D), lambda i, c: (i, 0)),
            pl.BlockSpec((1, NBLK, D), lambda i, c: (c, i, 0)),
            pl.BlockSpec((NBLK, D), lambda i, c: (i, 0)),
        ],
        out_shape=[
            jax.ShapeDtypeStruct((N, D), jnp.float32),
            jax.ShapeDtypeStruct((N, D), jnp.float32),
            jax.ShapeDtypeStruct((2, N, D), jnp.float32),
            jax.ShapeDtypeStruct((N, D), jnp.float32),
        ],
    )(hxp, ahp, acc0, acc1, aw, ab, bw, bb, dw, db, ew, eb)


@jax.jit
def _k_ce(ex0, ex1, cw, cb):
    grid = (E // EBLK,)
    return pl.pallas_call(
        _ce_body,
        grid=grid,
        in_specs=[
            pl.BlockSpec((EBLK, H), lambda i: (i, 0)),
            pl.BlockSpec((EBLK, H), lambda i: (i, 0)),
            pl.BlockSpec((D, D), lambda i: (0, 0)),
            pl.BlockSpec((1, D), lambda i: (0, 0)),
        ],
        out_specs=pl.BlockSpec((2, EBLK, H), lambda i: (0, i, 0)),
        out_shape=jax.ShapeDtypeStruct((2, E, H), jnp.float32),
    )(ex0, ex1, cw, cb)


@jax.jit
def _k_post(hxp, ahp, acc0, acc1, wo, bo):
    grid = (N // NBLK,)
    return pl.pallas_call(
        _post_body,
        grid=grid,
        in_specs=[
            pl.BlockSpec((NBLK, D), lambda i: (i, 0)),
            pl.BlockSpec((NBLK, D), lambda i: (i, 0)),
            pl.BlockSpec((NBLK, D), lambda i: (i, 0)),
            pl.BlockSpec((NBLK, D), lambda i: (i, 0)),
            pl.BlockSpec((D, D), lambda i: (0, 0)),
            pl.BlockSpec((1, D), lambda i: (0, 0)),
        ],
        out_specs=pl.BlockSpec((NBLK, D), lambda i: (i, 0)),
        out_shape=jax.ShapeDtypeStruct((N, D), jnp.float32),
    )(hxp, ahp, acc0, acc1, wo, bo)


def _ex0_body(w_ref, c1_ref, c0_ref, we_ref, be_ref, ce_ref, e0_ref, e1_ref):
    w = w_ref[...]
    ce_ref[0, :, :] = w * c1_ref[0:1, :] + c0_ref[0:1, :]
    ce_ref[1, :, :] = w * c1_ref[1:2, :] + c0_ref[1:2, :]
    e0_ref[...] = w * we_ref[0:1, :] + be_ref[0:1, :]
    e1_ref[...] = w * we_ref[1:2, :] + be_ref[1:2, :]


@jax.jit
def _k_ex0(w, c1, c0, wev, bev):
    grid = (E // EBLK,)
    two64 = pl.BlockSpec((2, H), lambda i: (0, 0))
    return pl.pallas_call(
        _ex0_body,
        grid=grid,
        in_specs=[pl.BlockSpec((EBLK, 1), lambda i: (i, 0)),
                  two64, two64, two64, two64],
        out_specs=[
            pl.BlockSpec((2, EBLK, H), lambda i: (0, i, 0)),
            pl.BlockSpec((EBLK, H), lambda i: (i, 0)),
            pl.BlockSpec((EBLK, H), lambda i: (i, 0)),
        ],
        out_shape=[
            jax.ShapeDtypeStruct((2, E, H), jnp.float32),
            jax.ShapeDtypeStruct((E, H), jnp.float32),
            jax.ShapeDtypeStruct((E, H), jnp.float32),
        ],
    )(w, c1, c0, wev, bev)


NACC = 5008
NEP = 2
SC_NW2 = 5


def _make_sc_layer():
    e_per_s = E // SC_NS
    nblk = e_per_s // SC_B
    nhalf = N // NEP
    nwr = nhalf // SC_NW2
    mesh = plsc.VectorSubcoreMesh(core_axis_name="c", subcore_axis_name="s",
                                  num_cores=1)

    out_type = [jax.ShapeDtypeStruct((N, D), jnp.float32),
                jax.ShapeDtypeStruct((N, D), jnp.float32),
                jax.ShapeDtypeStruct((E, H), jnp.float32),
                jax.ShapeDtypeStruct((E, H), jnp.float32),
                jax.ShapeDtypeStruct((E, D), jnp.float32)]

    scratch = [
        pltpu.VMEM((SC_B,), jnp.int32),
        pltpu.VMEM((SC_B,), jnp.int32),
        pltpu.VMEM((SC_B,), jnp.int32),
        pltpu.VMEM((SC_B, D), jnp.float32),
        pltpu.VMEM((SC_B, D), jnp.float32),
        pltpu.VMEM((SC_B, H), jnp.float32),
        pltpu.VMEM((SC_B, H), jnp.float32),
        pltpu.VMEM((SC_B, D), jnp.float32),
        pltpu.VMEM((ZROWS, D), jnp.float32),
        pltpu.VMEM_SHARED((NACC, D), jnp.float32),
        pltpu.SemaphoreType.DMA,
        pltpu.SemaphoreType.DMA,
        pltpu.SemaphoreType.DMA,
        pltpu.SemaphoreType.DMA,
    ]

    def body(src_h, dst_h, dbt_h, eht_h, ce_h, ex0_h, ex1_h,
             acc0_h, acc1_h, exo0_h, exo1_h, cs_h,
             src_v, dst_v, idx_v, db_buf, eh_buf, ce_buf, ex_buf,
             cs_buf, zbuf, acc_sp, sem1, sem2, sem3, sem4):
        s = lax.axis_index("s")
        base_e = s * e_per_s

        def zrow(r, _):
            for k in range(D // 16):
                zbuf[r, pl.ds(16 * k, 16)] = jnp.zeros((16,), jnp.float32)
            return 0

        lax.fori_loop(0, ZROWS, zrow, 0)

        def clamp_idx(lo):
            def idxc(k, _):
                sl = pl.ds(16 * k, 16)
                t = dst_v[sl] - lo
                ok = (t >= 0) & (t < nhalf)
                idx_v[sl] = jnp.where(ok, t, nhalf)
                return 0

            lax.fori_loop(0, SC_B // 16, idxc, 0)

        for half in (0, 1):
            ex_h = ex0_h if half == 0 else ex1_h
            exo_h = exo0_h if half == 0 else exo1_h
            acc_h = acc0_h if half == 0 else acc1_h

            for ep in range(NEP):
                lo = ep * nhalf

                @pl.when(s < SC_NW2)
                def _():
                    for i in range(nwr // ZROWS):
                        pltpu.sync_copy(
                            zbuf, acc_sp.at[pl.ds(s * nwr + ZROWS * i, ZROWS)])

                @pl.when(s == SC_NW2)
                def _():
                    pltpu.sync_copy(zbuf.at[pl.ds(0, 8)],
                                    acc_sp.at[pl.ds(nhalf, 8)])

                plsc.subcore_barrier()

                if ep == 0:
                    def eblk(b, _):
                        e0 = base_e + b * SC_B
                        pltpu.sync_copy(src_h.at[pl.ds(e0, SC_B)], src_v)
                        pltpu.sync_copy(dst_h.at[pl.ds(e0, SC_B)], dst_v)

                        if half == 0:
                            sidx = src_v
                        else:
                            def offc(k, _):
                                sl = pl.ds(16 * k, 16)
                                src_v[sl] = src_v[sl] + N
                                return 0

                            lax.fori_loop(0, SC_B // 16, offc, 0)
                            sidx = src_v
                        cp1 = pltpu.async_copy(dbt_h.at[sidx], db_buf, sem1)
                        cp2 = pltpu.async_copy(eht_h.at[dst_v], eh_buf, sem2)
                        cp3 = pltpu.async_copy(
                            ce_h.at[half, pl.ds(e0, SC_B)], ce_buf, sem3)
                        cp4 = pltpu.async_copy(
                            ex_h.at[pl.ds(e0, SC_B)], ex_buf, sem4)
                        clamp_idx(lo)
                        cp1.wait()
                        cp2.wait()
                        cp3.wait()
                        cp4.wait()

                        def row(r, _):
                            for k in range(H // 16):
                                sl = pl.ds(16 * k, 16)
                                dh = db_buf[r, sl]
                                bh = db_buf[r, pl.ds(H + 16 * k, 16)]
                                en = dh \
                                    + eh_buf[r, pl.ds(half * H + 16 * k, 16)] \
                                    + ce_buf[r, sl]
                                sg = 1.0 / (1.0 + jnp.exp(-en))
                                cs_buf[r, sl] = sg * bh
                                cs_buf[r, pl.ds(H + 16 * k, 16)] = sg
                                ex_buf[r, sl] = ex_buf[r, sl] \
                                    + jnp.maximum(en, 0.0)
                            return 0

                        lax.fori_loop(0, SC_B, row, 0)

                        pltpu.sync_copy(cs_buf, acc_sp.at[idx_v], add=True)
                        pltpu.sync_copy(cs_buf, cs_h.at[pl.ds(e0, SC_B)])
                        pltpu.sync_copy(ex_buf, exo_h.at[pl.ds(e0, SC_B)])
                        return 0
                else:
                    def eblk(b, _):
                        e0 = base_e + b * SC_B
                        cp3 = pltpu.async_copy(
                            cs_h.at[pl.ds(e0, SC_B)], cs_buf, sem3)
                        pltpu.sync_copy(dst_h.at[pl.ds(e0, SC_B)], dst_v)
                        clamp_idx(lo)
                        cp3.wait()
                        pltpu.sync_copy(cs_buf, acc_sp.at[idx_v], add=True)
                        return 0

                lax.fori_loop(0, nblk, eblk, 0)
                plsc.subcore_barrier()

                @pl.when(s < SC_NW2)
                def _():
                    pltpu.sync_copy(acc_sp.at[pl.ds(s * nwr, nwr)],
                                    acc_h.at[pl.ds(lo + s * nwr, nwr)])

                plsc.subcore_barrier()

    return pl.kernel(body, out_type=out_type, mesh=mesh,
                     scratch_types=scratch)


_SC_EDGE = _make_sc_layer()


@jax.jit
def _sc_layer(src, dst, dbt, eht, ce, ex0, ex1):
    a0, a1, eo0, eo1, _ = _SC_EDGE(src, dst, dbt, eht, ce, ex0, ex1)
    return a0, a1, eo0, eo1


def kernel(h, edge_index, edge_weight, W_h, b_h, W_e, b_e, Aw, Ab, Bw, Bb,
           Cw, Cb, Dw, Db, Ew, Eb, W_out, b_out):
    src = edge_index[0]
    dst = edge_index[1]
    r = lambda v: v.reshape(1, D)

    hx0, ah0, dbt0, eht0 = _k_pre(h, W_h, r(b_h), Aw[0], r(Ab[0]), Bw[0],
                                  r(Bb[0]), Dw[0], r(Db[0]), Ew[0], r(Eb[0]))

    wrow = W_e[0]
    c1 = (wrow @ Cw[0]).reshape(2, H)
    c0 = (b_e @ Cw[0] + Cb[0]).reshape(2, H)
    wev = wrow.reshape(2, H)
    bev = b_e.reshape(2, H)
    ce0, ex00, ex01 = _k_ex0(edge_weight.reshape(E, 1), c1, c0, wev, bev)

    acc00, acc01, ex10, ex11 = _sc_layer(src, dst, dbt0.reshape(2 * N, D),
                                         eht0, ce0, ex00, ex01)

    hx1, ah1, dbt1, eht1 = _k_node(hx0, ah0, acc00, acc01, Aw[1], r(Ab[1]),
                                   Bw[1], r(Bb[1]), Dw[1], r(Db[1]),
                                   Ew[1], r(Eb[1]))
    ce1 = _k_ce(ex10, ex11, Cw[1], r(Cb[1]))
    acc10, acc11, ex20, ex21 = _sc_layer(src, dst, dbt1.reshape(2 * N, D),
                                         eht1, ce1, ex10, ex11)

    hx2, ah2, dbt2, eht2 = _k_node(hx1, ah1, acc10, acc11, Aw[2], r(Ab[2]),
                                   Bw[2], r(Bb[2]), Dw[2], r(Db[2]),
                                   Ew[2], r(Eb[2]))
    ce2 = _k_ce(ex20, ex21, Cw[2], r(Cb[2]))
    acc20, acc21, _, _ = _sc_layer(src, dst, dbt2.reshape(2 * N, D),
                                   eht2, ce2, ex20, ex21)

    wo = jnp.pad(W_out, ((0, 0), (0, D - NC)))
    bo = jnp.pad(b_out, (0, D - NC)).reshape(1, D)
    out = _k_post(hx2, ah2, acc20, acc21, wo, bo)
    return out[:, :NC]


# speedup vs baseline: 1.1478x; 1.0012x over previous
import jax
import jax.numpy as jnp
from jax import lax
from jax.experimental import pallas as pl
from jax.experimental.pallas import tpu as pltpu
from jax.experimental.pallas import tpu_sc as plsc

N = 10000
E = 320000
D = 128
H = 64
NC = 40

NBLK = 1000
EBLK = 1000
SC_NS = 16
SC_B = 80
SC_NW = 10
NWR = N // SC_NW
ZROWS = 200


def _mm(x, w, b):
    return jnp.dot(x, w, preferred_element_type=jnp.float32) + b


def _pre_body(h_ref, wh, bh, aw, ab, bw, bb, dw, db, ew, eb,
              hx_ref, ah_ref, dbt_ref, eht_ref):
    c = pl.program_id(1)
    hx = _mm(h_ref[...], wh[...], bh[...])
    hx_ref[...] = hx
    ah_ref[...] = _mm(hx, aw[...], ab[...])
    dh = _mm(hx, dw[...], db[...])
    bhx = _mm(hx, bw[...], bb[...])
    eh = _mm(hx, ew[...], eb[...])

    eht_ref[...] = eh

    @pl.when(c == 0)
    def _():
        dbt_ref[0, :, 0:H] = dh[:, 0:H]
        dbt_ref[0, :, H:D] = bhx[:, 0:H]

    @pl.when(c == 1)
    def _():
        dbt_ref[0, :, 0:H] = dh[:, H:D]
        dbt_ref[0, :, H:D] = bhx[:, H:D]


def _node_body(hxp_ref, ahp_ref, acc0_ref, acc1_ref, aw, ab, bw, bb, dw, db,
               ew, eb, hx_ref, ah_ref, dbt_ref, eht_ref):
    c = pl.program_id(1)
    num = jnp.concatenate([acc0_ref[:, 0:H], acc1_ref[:, 0:H]], axis=1)
    den = jnp.concatenate([acc0_ref[:, H:D], acc1_ref[:, H:D]], axis=1) + 1e-6
    hx = hxp_ref[...] + jnp.maximum(ahp_ref[...] + num / den, 0.0)
    hx_ref[...] = hx
    ah_ref[...] = _mm(hx, aw[...], ab[...])
    dh = _mm(hx, dw[...], db[...])
    bhx = _mm(hx, bw[...], bb[...])
    eh = _mm(hx, ew[...], eb[...])

    eht_ref[...] = eh

    @pl.when(c == 0)
    def _():
        dbt_ref[0, :, 0:H] = dh[:, 0:H]
        dbt_ref[0, :, H:D] = bhx[:, 0:H]

    @pl.when(c == 1)
    def _():
        dbt_ref[0, :, 0:H] = dh[:, H:D]
        dbt_ref[0, :, H:D] = bhx[:, H:D]


def _ce_body(ex0_ref, ex1_ref, cw, cb, ce_ref):
    x = jnp.concatenate([ex0_ref[...], ex1_ref[...]], axis=1)
    y = _mm(x, cw[...], cb[...])
    ce_ref[0, :, :] = y[:, 0:H]
    ce_ref[1, :, :] = y[:, H:D]


def _post_body(hxp_ref, ahp_ref, acc0_ref, acc1_ref, wo, bo, out_ref):
    num = jnp.concatenate([acc0_ref[:, 0:H], acc1_ref[:, 0:H]], axis=1)
    den = jnp.concatenate([acc0_ref[:, H:D], acc1_ref[:, H:D]], axis=1) + 1e-6
    hx = hxp_ref[...] + jnp.maximum(ahp_ref[...] + num / den, 0.0)
    logits = _mm(hx, wo[...], bo[...])
    mask = lax.broadcasted_iota(jnp.int32, logits.shape, 1) < NC
    neg = jnp.where(mask, logits, -1e30)
    m = jnp.max(neg, axis=1, keepdims=True)
    ez = jnp.where(mask, jnp.exp(logits - m), 0.0)
    lse = jnp.log(jnp.sum(ez, axis=1, keepdims=True))
    out_ref[...] = logits - m - lse


def _wspec():
    return pl.BlockSpec((D, D), lambda i, c: (0, 0))


def _bspec():
    return pl.BlockSpec((1, D), lambda i, c: (0, 0))


@jax.jit
def _k_pre(h, wh, bh, aw, ab, bw, bb, dw, db, ew, eb):
    grid = (N // NBLK, 2)
    return pl.pallas_call(
        _pre_body,
        grid=grid,
        in_specs=[pl.BlockSpec((NBLK, D), lambda i, c: (i, 0))]
        + [_wspec() if k % 2 == 0 else _bspec() for k in range(10)],
        out_specs=[
            pl.BlockSpec((NBLK, D), lambda i, c: (i, 0)),
            pl.BlockSpec((NBLK, D), lambda i, c: (i, 0)),
            pl.BlockSpec((1, NBLK, D), lambda i, c: (c, i, 0)),
            pl.BlockSpec((NBLK, D), lambda i, c: (i, 0)),
        ],
        out_shape=[
            jax.ShapeDtypeStruct((N, D), jnp.float32),
            jax.ShapeDtypeStruct((N, D), jnp.float32),
            jax.ShapeDtypeStruct((2, N, D), jnp.float32),
            jax.ShapeDtypeStruct((N, D), jnp.float32),
        ],
    )(h, wh, bh, aw, ab, bw, bb, dw, db, ew, eb)


@jax.jit
def _k_node(hxp, ahp, acc0, acc1, aw, ab, bw, bb, dw, db, ew, eb):
    grid = (N // NBLK, 2)
    return pl.pallas_call(
        _node_body,
        grid=grid,
        in_specs=[
            pl.BlockSpec((NBLK, D), lambda i, c: (i, 0)),
            pl.BlockSpec((NBLK, D), lambda i, c: (i, 0)),
            pl.BlockSpec((NBLK, D), lambda i, c: (i, 0)),
            pl.BlockSpec((NBLK, D), lambda i, c: (i, 0)),
        ]
        + [_wspec() if k % 2 == 0 else _bspec() for k in range(8)],
        out_specs=[
            pl.BlockSpec((NBLK, D), lambda i, c: (i, 0)),
            pl.BlockSpec((NBLK, D), lambda i, c: (i, 0)),
            pl.BlockSpec((1, NBLK, D), lambda i, c: (c, i, 0)),
            pl.BlockSpec((NBLK, D), lambda i, c: (i, 0)),
        ],
        out_shape=[
            jax.ShapeDtypeStruct((N, D), jnp.float32),
            jax.ShapeDtypeStruct((N, D), jnp.float32),
            jax.ShapeDtypeStruct((2, N, D), jnp.float32),
            jax.ShapeDtypeStruct((N, D), jnp.float32),
        ],
    )(hxp, ahp, acc0, acc1, aw, ab, bw, bb, dw, db, ew, eb)


@jax.jit
def _k_ce(ex0, ex1, cw, cb):
    grid = (E // EBLK,)
    return pl.pallas_call(
        _ce_body,
        grid=grid,
        in_specs=[
            pl.BlockSpec((EBLK, H), lambda i: (i, 0)),
            pl.BlockSpec((EBLK, H), lambda i: (i, 0)),
            pl.BlockSpec((D, D), lambda i: (0, 0)),
            pl.BlockSpec((1, D), lambda i: (0, 0)),
        ],
        out_specs=pl.BlockSpec((2, EBLK, H), lambda i: (0, i, 0)),
        out_shape=jax.ShapeDtypeStruct((2, E, H), jnp.float32),
    )(ex0, ex1, cw, cb)


@jax.jit
def _k_post(hxp, ahp, acc0, acc1, wo, bo):
    grid = (N // NBLK,)
    return pl.pallas_call(
        _post_body,
        grid=grid,
        in_specs=[
            pl.BlockSpec((NBLK, D), lambda i: (i, 0)),
            pl.BlockSpec((NBLK, D), lambda i: (i, 0)),
            pl.BlockSpec((NBLK, D), lambda i: (i, 0)),
            pl.BlockSpec((NBLK, D), lambda i: (i, 0)),
            pl.BlockSpec((D, D), lambda i: (0, 0)),
            pl.BlockSpec((1, D), lambda i: (0, 0)),
        ],
        out_specs=pl.BlockSpec((NBLK, D), lambda i: (i, 0)),
        out_shape=jax.ShapeDtypeStruct((N, D), jnp.float32),
    )(hxp, ahp, acc0, acc1, wo, bo)


def _ex0_body(w_ref, c1_ref, c0_ref, we_ref, be_ref, ce_ref, e0_ref, e1_ref):
    w = w_ref[...]
    ce_ref[0, :, :] = w * c1_ref[0:1, :] + c0_ref[0:1, :]
    ce_ref[1, :, :] = w * c1_ref[1:2, :] + c0_ref[1:2, :]
    e0_ref[...] = w * we_ref[0:1, :] + be_ref[0:1, :]
    e1_ref[...] = w * we_ref[1:2, :] + be_ref[1:2, :]


@jax.jit
def _k_ex0(w, c1, c0, wev, bev):
    grid = (E // EBLK,)
    two64 = pl.BlockSpec((2, H), lambda i: (0, 0))
    return pl.pallas_call(
        _ex0_body,
        grid=grid,
        in_specs=[pl.BlockSpec((EBLK, 1), lambda i: (i, 0)),
                  two64, two64, two64, two64],
        out_specs=[
            pl.BlockSpec((2, EBLK, H), lambda i: (0, i, 0)),
            pl.BlockSpec((EBLK, H), lambda i: (i, 0)),
            pl.BlockSpec((EBLK, H), lambda i: (i, 0)),
        ],
        out_shape=[
            jax.ShapeDtypeStruct((2, E, H), jnp.float32),
            jax.ShapeDtypeStruct((E, H), jnp.float32),
            jax.ShapeDtypeStruct((E, H), jnp.float32),
        ],
    )(w, c1, c0, wev, bev)


NACC = 5008
NEP = 2
SC_NW2 = 5


def _make_sc_layer():
    e_per_s = E // SC_NS
    nblk = e_per_s // SC_B
    nhalf = N // NEP
    nwr = nhalf // SC_NW2
    mesh = plsc.VectorSubcoreMesh(core_axis_name="c", subcore_axis_name="s",
                                  num_cores=1)

    out_type = [jax.ShapeDtypeStruct((N, D), jnp.float32),
                jax.ShapeDtypeStruct((N, D), jnp.float32),
                jax.ShapeDtypeStruct((E, H), jnp.float32),
                jax.ShapeDtypeStruct((E, H), jnp.float32),
                jax.ShapeDtypeStruct((E, D), jnp.float32)]

    scratch = [
        pltpu.VMEM((SC_B,), jnp.int32),
        pltpu.VMEM((SC_B,), jnp.int32),
        pltpu.VMEM((SC_B,), jnp.int32),
        pltpu.VMEM((SC_B, D), jnp.float32),
        pltpu.VMEM((SC_B, D), jnp.float32),
        pltpu.VMEM((SC_B, H), jnp.float32),
        pltpu.VMEM((SC_B, H), jnp.float32),
        pltpu.VMEM((SC_B, D), jnp.float32),
        pltpu.VMEM((ZROWS, D), jnp.float32),
        pltpu.VMEM_SHARED((NACC, D), jnp.float32),
        pltpu.SemaphoreType.DMA,
        pltpu.SemaphoreType.DMA,
        pltpu.SemaphoreType.DMA,
        pltpu.SemaphoreType.DMA,
    ]

    def body(src_h, dst_h, dbt_h, eht_h, ce_h, ex0_h, ex1_h,
             acc0_h, acc1_h, exo0_h, exo1_h, cs_h,
             src_v, dst_v, idx_v, db_buf, eh_buf, ce_buf, ex_buf,
             cs_buf, zbuf, acc_sp, sem1, sem2, sem3, sem4):
        s = lax.axis_index("s")
        base_e = s * e_per_s

        def zrow(r, _):
            for k in range(D // 16):
                zbuf[r, pl.ds(16 * k, 16)] = jnp.zeros((16,), jnp.float32)
            return 0

        lax.fori_loop(0, ZROWS, zrow, 0)

        def clamp_idx(lo):
            def idxc(k, _):
                sl = pl.ds(16 * k, 16)
                t = dst_v[sl] - lo
                ok = (t >= 0) & (t < nhalf)
                idx_v[sl] = jnp.where(ok, t, nhalf)
                return 0

            lax.fori_loop(0, SC_B // 16, idxc, 0)

        for half in (0, 1):
            ex_h = ex0_h if half == 0 else ex1_h
            exo_h = exo0_h if half == 0 else exo1_h
            acc_h = acc0_h if half == 0 else acc1_h

            for ep in range(NEP):
                lo = ep * nhalf

                @pl.when(s < SC_NW2)
                def _():
                    for i in range(nwr // ZROWS):
                        pltpu.sync_copy(
                            zbuf, acc_sp.at[pl.ds(s * nwr + ZROWS * i, ZROWS)])

                @pl.when(s == SC_NW2)
                def _():
                    pltpu.sync_copy(zbuf.at[pl.ds(0, 8)],
                                    acc_sp.at[pl.ds(nhalf, 8)])

                plsc.subcore_barrier()

                if ep == 0:
                    def eblk(b, _):
                        e0 = base_e + b * SC_B
                        pltpu.sync_copy(src_h.at[pl.ds(e0, SC_B)], src_v)
                        pltpu.sync_copy(dst_h.at[pl.ds(e0, SC_B)], dst_v)

                        if half == 0:
                            sidx = src_v
                        else:
                            def offc(k, _):
                                sl = pl.ds(16 * k, 16)
                                src_v[sl] = src_v[sl] + N
                                return 0

                            lax.fori_loop(0, SC_B // 16, offc, 0)
                            sidx = src_v
                        cp1 = pltpu.async_copy(dbt_h.at[sidx], db_buf, sem1)
                        cp2 = pltpu.async_copy(eht_h.at[dst_v], eh_buf, sem2)
                        cp3 = pltpu.async_copy(
                            ce_h.at[half, pl.ds(e0, SC_B)], ce_buf, sem3)
                        cp4 = pltpu.async_copy(
                            ex_h.at[pl.ds(e0, SC_B)], ex_buf, sem4)
                        clamp_idx(lo)
                        cp1.wait()
                        cp2.wait()
                        cp3.wait()
                        cp4.wait()

                        def row(r, _):
                            for k in range(H // 16):
                                sl = pl.ds(16 * k, 16)
                                dh = db_buf[r, sl]
                                bh = db_buf[r, pl.ds(H + 16 * k, 16)]
                                en = dh \
                                    + eh_buf[r, pl.ds(half * H + 16 * k, 16)] \
                                    + ce_buf[r, sl]
                                sg = 1.0 / (1.0 + jnp.exp(-en))
                                cs_buf[r, sl] = sg * bh
                                cs_buf[r, pl.ds(H + 16 * k, 16)] = sg
                                ex_buf[r, sl] = ex_buf[r, sl] \
                                    + jnp.maximum(en, 0.0)
                            return 0

                        lax.fori_loop(0, SC_B, row, 0)

                        pltpu.sync_copy(cs_buf, acc_sp.at[idx_v], add=True)
                        pltpu.sync_copy(cs_buf, cs_h.at[pl.ds(e0, SC_B)])
                        pltpu.sync_copy(ex_buf, exo_h.at[pl.ds(e0, SC_B)])
                        return 0
                else:
                    def eblk(b, _):
                        e0 = base_e + b * SC_B
                        cp3 = pltpu.async_copy(
                            cs_h.at[pl.ds(e0, SC_B)], cs_buf, sem3)
                        pltpu.sync_copy(dst_h.at[pl.ds(e0, SC_B)], dst_v)
                        clamp_idx(lo)
                        cp3.wait()
                        pltpu.sync_copy(cs_buf, acc_sp.at[idx_v], add=True)
                        return 0

                lax.fori_loop(0, nblk, eblk, 0)
                plsc.subcore_barrier()

                @pl.when(s < SC_NW2)
                def _():
                    pltpu.sync_copy(acc_sp.at[pl.ds(s * nwr, nwr)],
                                    acc_h.at[pl.ds(lo + s * nwr, nwr)])

                plsc.subcore_barrier()

    return pl.kernel(body, out_type=out_type, mesh=mesh,
                     scratch_types=scratch)


_SC_EDGE_CACHE = []


def _sc_edge():
    if not _SC_EDGE_CACHE:
        _SC_EDGE_CACHE.append(_make_sc_layer())
    return _SC_EDGE_CACHE[0]


@jax.jit
def _sc_layer(src, dst, dbt, eht, ce, ex0, ex1):
    a0, a1, eo0, eo1, _ = _sc_edge()(src, dst, dbt, eht, ce, ex0, ex1)
    return a0, a1, eo0, eo1


def kernel(h, edge_index, edge_weight, W_h, b_h, W_e, b_e, Aw, Ab, Bw, Bb,
           Cw, Cb, Dw, Db, Ew, Eb, W_out, b_out):
    src = edge_index[0]
    dst = edge_index[1]
    r = lambda v: v.reshape(1, D)

    hx0, ah0, dbt0, eht0 = _k_pre(h, W_h, r(b_h), Aw[0], r(Ab[0]), Bw[0],
                                  r(Bb[0]), Dw[0], r(Db[0]), Ew[0], r(Eb[0]))

    wrow = W_e[0]
    c1 = (wrow @ Cw[0]).reshape(2, H)
    c0 = (b_e @ Cw[0] + Cb[0]).reshape(2, H)
    wev = wrow.reshape(2, H)
    bev = b_e.reshape(2, H)
    ce0, ex00, ex01 = _k_ex0(edge_weight.reshape(E, 1), c1, c0, wev, bev)

    acc00, acc01, ex10, ex11 = _sc_layer(src, dst, dbt0.reshape(2 * N, D),
                                         eht0, ce0, ex00, ex01)

    hx1, ah1, dbt1, eht1 = _k_node(hx0, ah0, acc00, acc01, Aw[1], r(Ab[1]),
                                   Bw[1], r(Bb[1]), Dw[1], r(Db[1]),
                                   Ew[1], r(Eb[1]))
    ce1 = _k_ce(ex10, ex11, Cw[1], r(Cb[1]))
    acc10, acc11, ex20, ex21 = _sc_layer(src, dst, dbt1.reshape(2 * N, D),
                                         eht1, ce1, ex10, ex11)

    hx2, ah2, dbt2, eht2 = _k_node(hx1, ah1, acc10, acc11, Aw[2], r(Ab[2]),
                                   Bw[2], r(Bb[2]), Dw[2], r(Db[2]),
                                   Ew[2], r(Eb[2]))
    ce2 = _k_ce(ex20, ex21, Cw[2], r(Cb[2]))
    acc20, acc21, _, _ = _sc_layer(src, dst, dbt2.reshape(2 * N, D),
                                   eht2, ce2, ex20, ex21)

    wo = jnp.pad(W_out, ((0, 0), (0, D - NC)))
    bo = jnp.pad(b_out, (0, D - NC)).reshape(1, D)
    out = _k_post(hx2, ah2, acc20, acc21, wo, bo)
    return out[:, :NC]
